# baseline (speedup 1.0000x reference)
import functools
import math

import jax
import jax.numpy as jnp
import numpy as np
from jax import lax
from jax.experimental import pallas as pl
from jax.experimental.pallas import tpu as pltpu
from jax.experimental.pallas import tpu_sc as plsc

_D = 128
_SEQ = 200
_BATCH = 1024
_TOKENS = _BATCH * _SEQ
_VOCAB = 1000
_SCALE = math.sqrt(float(_D))

_info = plsc.get_sparse_core_info()
_NC, _NS = _info.num_cores, _info.num_subcores
_NW = _NC * _NS
_SEQ_PER_W = _BATCH // _NW
_CK = 4
_CP = 40
_NSQ = _SEQ_PER_W // _CK
_NPG = _SEQ // _CP
_NCHUNK = _NPG * _NSQ
_CTOK = _CK * _CP
_IDXW = _CTOK // 2
_IDXLEN = _NCHUNK * _IDXW + 16


def _positional(seq, d):
    pe = np.zeros((seq, d), dtype=np.float32)
    position = np.arange(0, seq, dtype=np.float32)[:, None]
    div_term = np.exp(
        np.arange(0, d, 2, dtype=np.float32) * (-math.log(10000.0) / d))
    pe[:, 0::2] = np.sin(position * div_term)
    pe[:, 1::2] = np.cos(position * div_term)
    return pe


_mesh = plsc.VectorSubcoreMesh(core_axis_name="c", subcore_axis_name="s")


@functools.partial(
    pl.kernel,
    out_type=jax.ShapeDtypeStruct((_TOKENS, _D), jnp.float32),
    mesh=_mesh,
    scratch_types=[
        pltpu.VMEM((_IDXLEN,), jnp.int32),
        pltpu.VMEM((_VOCAB // 2, _D), jnp.int32),
        pltpu.VMEM((_CTOK, _D), jnp.float32),
        pltpu.VMEM((_CTOK, _D), jnp.float32),
        pltpu.VMEM((_SEQ // 2, _D), jnp.int32),
        pltpu.SemaphoreType.DMA,
        pltpu.SemaphoreType.DMA,
    ],
)
def _emb_kernel(xprep_hbm, tabp_hbm, pe_hbm, out_hbm,
                idx_v, tab_v, fb0, fb1, pe_v, s0, s1):
    wid = lax.axis_index("s") * _NC + lax.axis_index("c")
    seq0 = wid * _SEQ_PER_W
    pltpu.sync_copy(xprep_hbm.at[wid], idx_v)
    pltpu.sync_copy(tabp_hbm, tab_v)
    pltpu.sync_copy(pe_hbm, pe_v)

    fbs = (fb0, fb1)
    ssems = (s0, s1)
    mask_hi = jnp.int32(-65536)

    def fire_scatter(c, fb, ssem):
        sq = c & (_NSQ - 1)
        pg = c >> 3
        for k in range(_CK):
            row = (seq0 + sq * _CK + k) * _SEQ + pg * _CP
            pltpu.async_copy(
                fb.at[pl.ds(k * _CP, _CP)],
                out_hbm.at[pl.ds(row, _CP)], ssem)

    def wait_scatter(fb, ssem):
        pltpu.make_async_copy(fb, out_hbm.at[pl.ds(0, _CTOK)], ssem).wait()

    def compute(c, fb):
        pe0 = (c >> 3) * (_CP // 2)
        idx_base = c * _IDXW

        def s_body(s, carry):
            tokv = idx_v[pl.ds(idx_base + 2 * s, 16)]
            pe_row = []
            pehalf = (s & 1) * (_D // 2)
            for m in range(_D // 32):
                pw = pe_v[pe0 + (s >> 1), pl.ds(pehalf + 16 * m, 16)]
                pe_row.append(jax.lax.bitcast_convert_type(pw << 16, jnp.float32))
                pe_row.append(jax.lax.bitcast_convert_type(pw, jnp.float32))
            for k in range(_CK):
                tok = (tokv[k // 2] >> (16 * (k % 2))) & 65535
                r = k * _CP + s
                half = (tok & 1) * (_D // 2)
                for m in range(_D // 32):
                    w32 = tab_v[tok >> 1, pl.ds(half + 16 * m, 16)]
                    a = jax.lax.bitcast_convert_type(w32 << 16, jnp.float32)
                    b = jax.lax.bitcast_convert_type(w32 & mask_hi, jnp.float32)
                    fb[r, pl.ds(32 * m, 16)] = a * _SCALE + pe_row[2 * m]
                    fb[r, pl.ds(32 * m + 16, 16)] = (
                        b * _SCALE + pe_row[2 * m + 1])
            return carry

        lax.fori_loop(0, _CP, s_body, 0)

    compute(jnp.int32(0), fb0)
    fire_scatter(jnp.int32(0), fb0, s0)
    compute(jnp.int32(1), fb1)
    fire_scatter(jnp.int32(1), fb1, s1)

    def pair_body(tt, carry):
        c0 = 2 * tt
        wait_scatter(fb0, s0)
        compute(c0, fb0)
        fire_scatter(c0, fb0, s0)
        wait_scatter(fb1, s1)
        compute(c0 + 1, fb1)
        fire_scatter(c0 + 1, fb1, s1)
        return carry

    lax.fori_loop(1, _NCHUNK // 2, pair_body, 0)
    wait_scatter(fb0, s0)
    wait_scatter(fb1, s1)


def _pack_rows(arr):
    n = arr.shape[0]
    tb = arr.astype(jnp.bfloat16)
    bits = jax.lax.bitcast_convert_type(tb, jnp.uint16).astype(jnp.uint32)
    g = bits.reshape(n, _D // 32, 2, 16)
    packed = g[:, :, 0, :] | (g[:, :, 1, :] << 16)
    return jax.lax.bitcast_convert_type(
        packed.reshape(n // 2, _D), jnp.int32)


def kernel(x, table):
    xg = (x.astype(jnp.uint32)
          .reshape(_NW, _NSQ, _CK, _NPG, _CP)
          .transpose(0, 3, 1, 4, 2)
          .reshape(_NW, _NCHUNK, _CP, 2, 2))
    packed_idx = xg[..., 0] | (xg[..., 1] << 16)
    xprep = jax.lax.bitcast_convert_type(
        packed_idx.reshape(_NW, _NCHUNK * _IDXW), jnp.int32)
    xprep = jnp.pad(xprep, ((0, 0), (0, 16)))
    pe = _pack_rows(jnp.asarray(_positional(_SEQ, _D)))
    out = _emb_kernel(xprep, _pack_rows(table), pe)
    return out.reshape(_BATCH, _SEQ, _D)


# speedup vs baseline: 1.0909x; 1.0909x over previous
import functools
import math

import jax
import jax.numpy as jnp
import numpy as np
from jax import lax
from jax.experimental import pallas as pl
from jax.experimental.pallas import tpu as pltpu
from jax.experimental.pallas import tpu_sc as plsc

_D = 128
_SEQ = 200
_BATCH = 1024
_TOKENS = _BATCH * _SEQ
_VOCAB = 1000
_SCALE = math.sqrt(float(_D))

_info = plsc.get_sparse_core_info()
_NC, _NS = _info.num_cores, _info.num_subcores
_NW = _NC * _NS
_SEQ_PER_W = _BATCH // _NW
_CK = 4
_CP = 40
_NSQ = _SEQ_PER_W // _CK
_NPG = _SEQ // _CP
_NCHUNK = _NPG * _NSQ
_CTOK = _CK * _CP
_IDXW = _CTOK // 2
_IDXLEN = _NCHUNK * _IDXW + 16


def _positional(seq, d):
    pe = np.zeros((seq, d), dtype=np.float32)
    position = np.arange(0, seq, dtype=np.float32)[:, None]
    div_term = np.exp(
        np.arange(0, d, 2, dtype=np.float32) * (-math.log(10000.0) / d))
    pe[:, 0::2] = np.sin(position * div_term)
    pe[:, 1::2] = np.cos(position * div_term)
    return pe


_mesh = plsc.VectorSubcoreMesh(core_axis_name="c", subcore_axis_name="s")


@functools.partial(
    pl.kernel,
    out_type=jax.ShapeDtypeStruct((_TOKENS, _D), jnp.float32),
    mesh=_mesh,
    scratch_types=[
        pltpu.VMEM((_IDXLEN,), jnp.int32),
        pltpu.VMEM((_VOCAB * _D // 2,), jnp.int32),
        pltpu.VMEM((_CTOK, _D), jnp.float32),
        pltpu.VMEM((_CTOK, _D), jnp.float32),
        pltpu.VMEM((_SEQ // 2, _D), jnp.int32),
        pltpu.SemaphoreType.DMA,
        pltpu.SemaphoreType.DMA,
    ],
)
def _emb_kernel(xprep_hbm, tabp_hbm, pe_hbm, out_hbm,
                idx_v, tab_v, fb0, fb1, pe_v, s0, s1):
    wid = lax.axis_index("s") * _NC + lax.axis_index("c")
    seq0 = wid * _SEQ_PER_W
    pltpu.sync_copy(xprep_hbm.at[wid], idx_v)
    pltpu.sync_copy(tabp_hbm, tab_v)
    pltpu.sync_copy(pe_hbm, pe_v)

    mask_hi = jnp.int32(-65536)
    mask_lo = jnp.int32(65535)

    def fire_scatter(c, fb, ssem):
        sq = c & (_NSQ - 1)
        pg = c >> 3
        for k in range(_CK):
            row = (seq0 + sq * _CK + k) * _SEQ + pg * _CP
            pltpu.async_copy(
                fb.at[pl.ds(k * _CP, _CP)],
                out_hbm.at[pl.ds(row, _CP)], ssem)

    def wait_scatter(fb, ssem):
        pltpu.make_async_copy(fb, out_hbm.at[pl.ds(0, _CTOK)], ssem).wait()

    def compute(c, fb):
        pe0 = (c >> 3) * (_CP // 2)
        idx_base = c * _IDXW

        def decode_toks(s):
            tokv = idx_v[pl.ds(idx_base + 2 * s, 16)]
            out = []
            for k in range(_CK):
                w = tokv[k // 2]
                out.append(((w >> 16) & mask_lo) if k % 2 else (w & mask_lo))
            return tuple(out)

        def s_body(s, toks):
            nxt = decode_toks(s + 1)
            pe_row = []
            pehalf = (s & 1) * (_D // 2)
            for m in range(_D // 32):
                pw = pe_v[pe0 + (s >> 1), pl.ds(pehalf + 16 * m, 16)]
                pe_row.append(
                    jax.lax.bitcast_convert_type(pw << 16, jnp.float32))
                pe_row.append(jax.lax.bitcast_convert_type(pw, jnp.float32))
            for k in range(_CK):
                r = k * _CP + s
                for m in range(_D // 32):
                    w32 = tab_v[pl.ds(toks[k] + 16 * m, 16)]
                    a = jax.lax.bitcast_convert_type(w32 << 16, jnp.float32)
                    b = jax.lax.bitcast_convert_type(
                        w32 & mask_hi, jnp.float32)
                    fb[r, pl.ds(32 * m, 16)] = a * _SCALE + pe_row[2 * m]
                    fb[r, pl.ds(32 * m + 16, 16)] = (
                        b * _SCALE + pe_row[2 * m + 1])
            return nxt

        lax.fori_loop(0, _CP, s_body, decode_toks(jnp.int32(0)))

    compute(jnp.int32(0), fb0)
    fire_scatter(jnp.int32(0), fb0, s0)
    compute(jnp.int32(1), fb1)
    fire_scatter(jnp.int32(1), fb1, s1)

    def pair_body(tt, carry):
        c0 = 2 * tt
        wait_scatter(fb0, s0)
        compute(c0, fb0)
        fire_scatter(c0, fb0, s0)
        wait_scatter(fb1, s1)
        compute(c0 + 1, fb1)
        fire_scatter(c0 + 1, fb1, s1)
        return carry

    lax.fori_loop(1, _NCHUNK // 2 - 1, pair_body, 0)

    wait_scatter(fb0, s0)
    compute(jnp.int32(_NCHUNK - 2), fb0)
    fire_scatter(jnp.int32(_NCHUNK - 2), fb0, s0)
    wait_scatter(fb1, s1)
    compute(jnp.int32(_NCHUNK - 1), fb1)
    fire_scatter(jnp.int32(_NCHUNK - 1), fb1, s1)
    wait_scatter(fb0, s0)
    wait_scatter(fb1, s1)


def _pack_rows(arr):
    n = arr.shape[0]
    tb = arr.astype(jnp.bfloat16)
    bits = jax.lax.bitcast_convert_type(tb, jnp.uint16).astype(jnp.uint32)
    g = bits.reshape(n, _D // 32, 2, 16)
    packed = g[:, :, 0, :] | (g[:, :, 1, :] << 16)
    return jax.lax.bitcast_convert_type(
        packed.reshape(n // 2, _D), jnp.int32)


def kernel(x, table):
    xg = (x.astype(jnp.uint32)
          .reshape(_NW, _NSQ, _CK, _NPG, _CP)
          .transpose(0, 3, 1, 4, 2)
          .reshape(_NW, _NCHUNK, _CP, 2, 2)) * (_D // 2)
    packed_idx = xg[..., 0] | (xg[..., 1] << 16)
    xprep = jax.lax.bitcast_convert_type(
        packed_idx.reshape(_NW, _NCHUNK * _IDXW), jnp.int32)
    xprep = jnp.pad(xprep, ((0, 0), (0, 16)))
    pe = _pack_rows(jnp.asarray(_positional(_SEQ, _D)))
    tab = _pack_rows(table).reshape(_VOCAB * _D // 2)
    out = _emb_kernel(xprep, tab, pe)
    return out.reshape(_BATCH, _SEQ, _D)


# speedup vs baseline: 1.4475x; 1.3269x over previous
import functools
import math

import jax
import jax.numpy as jnp
import numpy as np
from jax import lax
from jax.experimental import pallas as pl
from jax.experimental.pallas import tpu as pltpu
from jax.experimental.pallas import tpu_sc as plsc

_D = 128
_SEQ = 200
_BATCH = 1024
_TOKENS = _BATCH * _SEQ
_SCALE = math.sqrt(float(_D))

_info = plsc.get_sparse_core_info()
_NC, _NS = _info.num_cores, _info.num_subcores
_NW = _NC * _NS
_SEQ_PER_W = _BATCH // _NW
_IDX_MINOR = 100
_NBUF = 3


def _positional(seq, d):
    pe = np.zeros((seq, d), dtype=np.float32)
    position = np.arange(0, seq, dtype=np.float32)[:, None]
    div_term = np.exp(
        np.arange(0, d, 2, dtype=np.float32) * (-math.log(10000.0) / d))
    pe[:, 0::2] = np.sin(position * div_term)
    pe[:, 1::2] = np.cos(position * div_term)
    return pe


_mesh = plsc.VectorSubcoreMesh(core_axis_name="c", subcore_axis_name="s")


@functools.partial(
    pl.kernel,
    out_type=jax.ShapeDtypeStruct((_TOKENS, _D), jnp.float32),
    mesh=_mesh,
    scratch_types=[
        pltpu.VMEM((2 * _SEQ_PER_W, _IDX_MINOR), jnp.int32),
        pltpu.VMEM((_SEQ, _D), jnp.float32),
        pltpu.VMEM((_SEQ, _D), jnp.float32),
        pltpu.VMEM((_SEQ, _D), jnp.float32),
        pltpu.VMEM((_SEQ, _D), jnp.float32),
        pltpu.SemaphoreType.DMA,
        pltpu.SemaphoreType.DMA,
        pltpu.SemaphoreType.DMA,
        pltpu.SemaphoreType.DMA,
        pltpu.SemaphoreType.DMA,
        pltpu.SemaphoreType.DMA,
    ],
)
def _emb_kernel(x_hbm, table_hbm, pe_hbm, out_hbm,
                idx_v, b0, b1, b2, pe_v, g0, g1, g2, s0, s1, s2):
    wid = lax.axis_index("s") * _NC + lax.axis_index("c")
    seq0 = wid * _SEQ_PER_W
    pltpu.sync_copy(x_hbm.at[pl.ds(seq0 * 2, 2 * _SEQ_PER_W)], idx_v)
    pltpu.sync_copy(pe_hbm, pe_v)

    bufs = (b0, b1, b2)
    gsems = (g0, g1, g2)
    ssems = (s0, s1, s2)

    def fire_gather(h):
        p = h % _NBUF
        c0 = pltpu.async_copy(
            table_hbm.at[idx_v.at[2 * h]],
            bufs[p].at[pl.ds(0, _IDX_MINOR)], gsems[p])
        c1 = pltpu.async_copy(
            table_hbm.at[idx_v.at[2 * h + 1]],
            bufs[p].at[pl.ds(_IDX_MINOR, _IDX_MINOR)], gsems[p])
        return (c0, c1)

    def fire_scatter(h):
        p = h % _NBUF
        return pltpu.async_copy(
            bufs[p], out_hbm.at[pl.ds((seq0 + h) * _SEQ, _SEQ)], ssems[p])

    def compute(h):
        buf = bufs[h % _NBUF]

        def row_body(s, c2):
            for j in range(_D // 16):
                sl = pl.ds(j * 16, 16)
                buf[s, sl] = buf[s, sl] * _SCALE + pe_v[s, sl]
            return c2

        lax.fori_loop(0, _SEQ, row_body, 0)

    gathers = {0: fire_gather(0), 1: fire_gather(1)}
    scatters = {}
    for h in range(_SEQ_PER_W):
        for c in gathers.pop(h):
            c.wait()
        compute(h)
        scatters[h] = fire_scatter(h)
        if h >= 1:
            scatters.pop(h - 1).wait()
        if h + 2 < _SEQ_PER_W:
            gathers[h + 2] = fire_gather(h + 2)
    scatters.pop(_SEQ_PER_W - 1).wait()


def kernel(x, table):
    x_flat = x.astype(jnp.int32).reshape(_TOKENS // _IDX_MINOR, _IDX_MINOR)
    pe = jnp.asarray(_positional(_SEQ, _D))
    out = _emb_kernel(x_flat, table, pe)
    return out.reshape(_BATCH, _SEQ, _D)
